# baseline (speedup 1.0000x reference)
import functools

import jax
import jax.numpy as jnp
from jax import lax
from jax.experimental import pallas as pl
from jax.experimental.pallas import tpu as pltpu
from jax.experimental.pallas import tpu_sc as plsc

VOCAB = 1000000
EMBED = 64
HIDDEN = 128
L = 16384

NC = 2
NS = 16
NW = NC * NS
PER_W = L // NW
CHUNK = 128
NCHUNK = PER_W // CHUNK

B = 51200
S = 6 * B
G = 14
P = S + G * B
SLICE = P // NS

HALF = S // NC
PTL = HALF // 2
CH = 1024
NCH = PTL // CH
NBUF = 4


@functools.partial(
    pl.kernel,
    mesh=plsc.VectorSubcoreMesh(core_axis_name="c", subcore_axis_name="s"),
    out_type=jax.ShapeDtypeStruct((NC, P), jnp.float32),
    scratch_types=[
        pltpu.VMEM((NCHUNK, CHUNK), jnp.int32),
        pltpu.VMEM((NCHUNK, CHUNK), jnp.float32),
        pltpu.VMEM_SHARED((P,), jnp.float32),
    ],
)
def _sc_counts(x_hbm, zeros_hbm, out_hbm, idx_v, ones_v, c_sh):
    cid = lax.axis_index("c")
    sid = lax.axis_index("s")
    wid = sid * NC + cid
    pltpu.sync_copy(x_hbm.at[wid], idx_v)
    for j in range(NCHUNK):
        for k in range(CHUNK // 16):
            ones_v[j, pl.ds(k * 16, 16)] = jnp.ones((16,), jnp.float32)
    pltpu.sync_copy(zeros_hbm.at[pl.ds(sid * SLICE, SLICE)],
                    c_sh.at[pl.ds(sid * SLICE, SLICE)])
    plsc.subcore_barrier()
    for j in range(NCHUNK):
        pltpu.sync_copy(ones_v.at[j], c_sh.at[idx_v.at[j]], add=True)
    plsc.subcore_barrier()
    pltpu.sync_copy(c_sh.at[pl.ds(sid * SLICE, SLICE)],
                    out_hbm.at[cid, pl.ds(sid * SLICE, SLICE)])


@functools.partial(
    pl.kernel,
    mesh=plsc.VectorSubcoreMesh(core_axis_name="c", subcore_axis_name="s"),
    out_type=jax.ShapeDtypeStruct((NC, NS, 8, 16), jnp.float32),
    scratch_types=(
        [pltpu.VMEM((8, CH), jnp.float32) for _ in range(NBUF)]
        + [pltpu.VMEM((2, CH), jnp.float32) for _ in range(NBUF)]
        + [pltpu.VMEM((8, 16), jnp.float32)]
        + [pltpu.SemaphoreType.DMA for _ in range(NBUF)]
    ),
)
def _sc_sweep(table_hbm, cnt_hbm, out_hbm, *scratch):
    tvs = scratch[:NBUF]
    cvs = scratch[NBUF:2 * NBUF]
    ov = scratch[2 * NBUF]
    sems = scratch[2 * NBUF + 1:]
    cid = lax.axis_index("c")
    sid = lax.axis_index("s")
    row0 = (sid // 2) * 8
    lane0 = cid * HALF + (sid % 2) * PTL

    def issue(k, b):
        l0 = lane0 + (k % NCH) * CH
        pltpu.async_copy(table_hbm.at[pl.ds(row0, 8), pl.ds(l0, CH)],
                         tvs[b], sems[b])
        pltpu.async_copy(cnt_hbm.at[:, pl.ds(l0, CH)], cvs[b], sems[b])

    def wait(b):
        pltpu.make_async_copy(table_hbm.at[pl.ds(0, 8), pl.ds(0, CH)],
                              tvs[b], sems[b]).wait()
        pltpu.make_async_copy(cnt_hbm.at[:, pl.ds(0, CH)], cvs[b],
                              sems[b]).wait()

    def process(b, accs):
        tv, cv = tvs[b], cvs[b]

        def m_body(m, a):
            c = cv[0, pl.ds(m * 16, 16)] + cv[1, pl.ds(m * 16, 16)]
            return tuple(a[e] + tv[e, pl.ds(m * 16, 16)] * c
                         for e in range(8))
        return lax.fori_loop(0, CH // 16, m_body, accs, unroll=8)

    for b in range(NBUF):
        issue(b, b)

    def outer(q, accs):
        for b in range(NBUF):
            k = q * NBUF + b
            wait(b)
            accs = process(b, accs)
            issue(k + NBUF, b)
        return accs

    zero = jnp.zeros((16,), jnp.float32)
    accs = lax.fori_loop(0, NCH // NBUF, outer, (zero,) * 8)
    for b in range(NCH % NBUF):
        wait(b)
        accs = process(b, accs)
    for b in range(NCH % NBUF, NBUF):
        wait(b)
    for e in range(8):
        ov[e, pl.ds(0, 16)] = accs[e]
    pltpu.sync_copy(ov, out_hbm.at[cid, sid])


def _matvec_body(tab_ref, cnt_ref, o_ref):
    i = pl.program_id(0)

    @pl.when(i == 0)
    def _():
        o_ref[...] = jnp.zeros_like(o_ref)

    c = cnt_ref[0:1, :] + cnt_ref[1:2, :]

    @pl.when(i < G - 1)
    def _():
        o_ref[...] += lax.dot_general(c, tab_ref[...],
                                      (((1,), (1,)), ((), ())),
                                      preferred_element_type=jnp.float32)

    @pl.when(i == G - 1)
    def _():
        lane = lax.broadcasted_iota(jnp.int32, (1, B), 1) + (i + S // B) * B
        tb = jnp.where(lane < VOCAB, tab_ref[...], 0.0)
        o_ref[...] += lax.dot_general(c, tb, (((1,), (1,)), ((), ())),
                                      preferred_element_type=jnp.float32)


def _combine_body(acc_ref, scp_ref, w1_ref, b1_ref, w2_ref, b2_ref, o_ref):
    v = scp_ref[...]
    le = lax.broadcasted_iota(jnp.int32, (128, EMBED), 0)
    ee = lax.broadcasted_iota(jnp.int32, (128, EMBED), 1)
    bsel = jnp.where((le // 16) == (ee % 8), 1.0, 0.0)
    w = lax.dot_general(v, bsel, (((1,), (0,)), ((), ())),
                        preferred_element_type=jnp.float32)
    ti = lax.broadcasted_iota(jnp.int32, (32, EMBED), 0)
    ee2 = lax.broadcasted_iota(jnp.int32, (32, EMBED), 1)
    a2 = jnp.where(((ti % NS) // 2) == (ee2 // 8), 1.0, 0.0)
    s_sc = jnp.sum(w * a2, axis=0, keepdims=True)
    e = (acc_ref[...] + s_sc) * (1.0 / L)
    h = lax.dot_general(e, w1_ref[...], (((1,), (1,)), ((), ())),
                        preferred_element_type=jnp.float32)
    h = jnp.maximum(h + b1_ref[...], 0.0)
    o_ref[...] = jnp.sum(h * w2_ref[...], axis=1, keepdims=True) + b2_ref[...]


def kernel(x, table, W1, b1, W2, b2):
    xi = x.astype(jnp.int32).reshape(NW, NCHUNK, CHUNK)
    zeros = jnp.zeros((P,), jnp.float32)
    counts = _sc_counts(xi, zeros)
    tableT = jnp.transpose(table)
    sc_part = _sc_sweep(tableT, counts).reshape(NW, 128)
    acc = pl.pallas_call(
        _matvec_body,
        grid=(G,),
        in_specs=[
            pl.BlockSpec((EMBED, B), lambda i: (0, i + S // B)),
            pl.BlockSpec((NC, B), lambda i: (0, i + S // B)),
        ],
        out_specs=pl.BlockSpec((1, EMBED), lambda i: (0, 0)),
        out_shape=jax.ShapeDtypeStruct((1, EMBED), jnp.float32),
    )(tableT, counts)
    out = pl.pallas_call(
        _combine_body,
        out_shape=jax.ShapeDtypeStruct((1, 1), jnp.float32),
    )(acc, sc_part, W1, b1.reshape(1, HIDDEN), W2, b2.reshape(1, 1))
    return out.reshape(1)


# speedup vs baseline: 1.0165x; 1.0165x over previous
import functools

import jax
import jax.numpy as jnp
from jax import lax
from jax.experimental import pallas as pl
from jax.experimental.pallas import tpu as pltpu
from jax.experimental.pallas import tpu_sc as plsc

VOCAB = 1000000
EMBED = 64
HIDDEN = 128
L = 16384

NC = 2
NS = 16
PER_T = L // NS
CHUNK = 128
NCHUNK = PER_T // CHUNK

B = 51200
S = 6 * B
G = 14
P = S + G * B
SLICE = P // NS

HALF = S // NC
PTL = HALF // 2
CH = 3072
NCH = PTL // CH
NBUF = 3


@functools.partial(
    pl.kernel,
    mesh=plsc.VectorSubcoreMesh(core_axis_name="c", subcore_axis_name="s"),
    out_type=jax.ShapeDtypeStruct((1, P), jnp.float32),
    scratch_types=[
        pltpu.VMEM((NCHUNK, CHUNK), jnp.int32),
        pltpu.VMEM((NCHUNK, CHUNK), jnp.float32),
        pltpu.VMEM_SHARED((P,), jnp.float32),
    ],
)
def _sc_counts(x_hbm, zeros_hbm, out_hbm, idx_v, ones_v, c_sh):
    cid = lax.axis_index("c")
    sid = lax.axis_index("s")

    @pl.when(cid == 0)
    def _():
        pltpu.sync_copy(x_hbm.at[sid], idx_v)
        for j in range(NCHUNK):
            for k in range(CHUNK // 16):
                ones_v[j, pl.ds(k * 16, 16)] = jnp.ones((16,), jnp.float32)
        pltpu.sync_copy(zeros_hbm.at[pl.ds(sid * SLICE, SLICE)],
                        c_sh.at[pl.ds(sid * SLICE, SLICE)])

    plsc.subcore_barrier()

    @pl.when(cid == 0)
    def _():
        for j in range(NCHUNK):
            pltpu.sync_copy(ones_v.at[j], c_sh.at[idx_v.at[j]], add=True)

    plsc.subcore_barrier()

    @pl.when(cid == 0)
    def _():
        pltpu.sync_copy(c_sh.at[pl.ds(sid * SLICE, SLICE)],
                        out_hbm.at[0, pl.ds(sid * SLICE, SLICE)])


@functools.partial(
    pl.kernel,
    mesh=plsc.VectorSubcoreMesh(core_axis_name="c", subcore_axis_name="s"),
    out_type=jax.ShapeDtypeStruct((NC, NS, 8, 16), jnp.float32),
    scratch_types=(
        [pltpu.VMEM((8, CH), jnp.float32) for _ in range(NBUF)]
        + [pltpu.VMEM((CH,), jnp.float32) for _ in range(NBUF)]
        + [pltpu.VMEM((8, 16), jnp.float32)]
        + [pltpu.SemaphoreType.DMA for _ in range(NBUF)]
    ),
)
def _sc_sweep(table_hbm, cnt_hbm, out_hbm, *scratch):
    tvs = scratch[:NBUF]
    cvs = scratch[NBUF:2 * NBUF]
    ov = scratch[2 * NBUF]
    sems = scratch[2 * NBUF + 1:]
    cid = lax.axis_index("c")
    sid = lax.axis_index("s")
    row0 = (sid // 2) * 8
    lane0 = cid * HALF + (sid % 2) * PTL

    def issue(k, b):
        l0 = lane0 + (k % NCH) * CH
        pltpu.async_copy(table_hbm.at[pl.ds(row0, 8), pl.ds(l0, CH)],
                         tvs[b], sems[b])
        pltpu.async_copy(cnt_hbm.at[0, pl.ds(l0, CH)], cvs[b], sems[b])

    def wait(b):
        pltpu.make_async_copy(table_hbm.at[pl.ds(0, 8), pl.ds(0, CH)],
                              tvs[b], sems[b]).wait()
        pltpu.make_async_copy(cnt_hbm.at[0, pl.ds(0, CH)], cvs[b],
                              sems[b]).wait()

    def process(b, accs):
        tv, cv = tvs[b], cvs[b]

        def m_body(m, a):
            c = cv[pl.ds(m * 16, 16)]
            return tuple(a[e] + tv[e, pl.ds(m * 16, 16)] * c
                         for e in range(8))
        return lax.fori_loop(0, CH // 16, m_body, accs, unroll=8)

    for b in range(NBUF):
        issue(b, b)

    def outer(q, accs):
        for b in range(NBUF):
            k = q * NBUF + b
            wait(b)
            accs = process(b, accs)
            issue(k + NBUF, b)
        return accs

    zero = jnp.zeros((16,), jnp.float32)
    accs = lax.fori_loop(0, NCH // NBUF, outer, (zero,) * 8)
    for b in range(NCH % NBUF):
        wait(b)
        accs = process(b, accs)
    for b in range(NCH % NBUF, NBUF):
        wait(b)
    for e in range(8):
        ov[e, pl.ds(0, 16)] = accs[e]
    pltpu.sync_copy(ov, out_hbm.at[cid, sid])


def _matvec_body(tab_ref, cnt_ref, o_ref):
    i = pl.program_id(0)

    @pl.when(i == 0)
    def _():
        o_ref[...] = jnp.zeros_like(o_ref)

    c = cnt_ref[...]

    @pl.when(i < G - 1)
    def _():
        o_ref[...] += lax.dot_general(c, tab_ref[...],
                                      (((1,), (1,)), ((), ())),
                                      preferred_element_type=jnp.float32)

    @pl.when(i == G - 1)
    def _():
        lane = lax.broadcasted_iota(jnp.int32, (1, B), 1) + (i + S // B) * B
        tb = jnp.where(lane < VOCAB, tab_ref[...], 0.0)
        o_ref[...] += lax.dot_general(c, tb, (((1,), (1,)), ((), ())),
                                      preferred_element_type=jnp.float32)


def _combine_body(acc_ref, scp_ref, w1_ref, b1_ref, w2_ref, b2_ref, o_ref):
    v = scp_ref[...]
    le = lax.broadcasted_iota(jnp.int32, (128, EMBED), 0)
    ee = lax.broadcasted_iota(jnp.int32, (128, EMBED), 1)
    bsel = jnp.where((le // 16) == (ee % 8), 1.0, 0.0)
    w = lax.dot_general(v, bsel, (((1,), (0,)), ((), ())),
                        preferred_element_type=jnp.float32)
    ti = lax.broadcasted_iota(jnp.int32, (32, EMBED), 0)
    ee2 = lax.broadcasted_iota(jnp.int32, (32, EMBED), 1)
    a2 = jnp.where(((ti % NS) // 2) == (ee2 // 8), 1.0, 0.0)
    s_sc = jnp.sum(w * a2, axis=0, keepdims=True)
    e = (acc_ref[...] + s_sc) * (1.0 / L)
    h = lax.dot_general(e, w1_ref[...], (((1,), (1,)), ((), ())),
                        preferred_element_type=jnp.float32)
    h = jnp.maximum(h + b1_ref[...], 0.0)
    o_ref[...] = jnp.sum(h * w2_ref[...], axis=1, keepdims=True) + b2_ref[...]


def kernel(x, table, W1, b1, W2, b2):
    xi = x.astype(jnp.int32).reshape(NS, NCHUNK, CHUNK)
    zeros = jnp.zeros((P,), jnp.float32)
    counts = _sc_counts(xi, zeros)
    tableT = jnp.transpose(table)
    sc_part = _sc_sweep(tableT, counts).reshape(NC * NS, 128)
    acc = pl.pallas_call(
        _matvec_body,
        grid=(G,),
        in_specs=[
            pl.BlockSpec((EMBED, B), lambda i: (0, i + S // B)),
            pl.BlockSpec((1, B), lambda i: (0, i + S // B)),
        ],
        out_specs=pl.BlockSpec((1, EMBED), lambda i: (0, 0)),
        out_shape=jax.ShapeDtypeStruct((1, EMBED), jnp.float32),
    )(tableT, counts)
    out = pl.pallas_call(
        _combine_body,
        out_shape=jax.ShapeDtypeStruct((1, 1), jnp.float32),
    )(acc, sc_part, W1, b1.reshape(1, HIDDEN), W2, b2.reshape(1, 1))
    return out.reshape(1)


# speedup vs baseline: 1.1422x; 1.1236x over previous
import functools

import jax
import jax.numpy as jnp
from jax import lax
from jax.experimental import pallas as pl
from jax.experimental.pallas import tpu as pltpu
from jax.experimental.pallas import tpu_sc as plsc

VOCAB = 1000000
EMBED = 64
HIDDEN = 128
L = 16384

NC = 2
NS = 16
PER_T = L // NS
CHUNK = 128
NCHUNK = PER_T // CHUNK

B = 51200
G = 20
P = G * B
SLICE = P // NS


@functools.partial(
    pl.kernel,
    mesh=plsc.VectorSubcoreMesh(core_axis_name="c", subcore_axis_name="s"),
    out_type=jax.ShapeDtypeStruct((1, P), jnp.float32),
    scratch_types=[
        pltpu.VMEM((NCHUNK, CHUNK), jnp.int32),
        pltpu.VMEM((NCHUNK, CHUNK), jnp.float32),
        pltpu.VMEM_SHARED((P,), jnp.float32),
    ],
)
def _sc_counts(x_hbm, zeros_hbm, out_hbm, idx_v, ones_v, c_sh):
    cid = lax.axis_index("c")
    sid = lax.axis_index("s")

    @pl.when(cid == 0)
    def _():
        pltpu.sync_copy(x_hbm.at[sid], idx_v)
        for j in range(NCHUNK):
            for k in range(CHUNK // 16):
                ones_v[j, pl.ds(k * 16, 16)] = jnp.ones((16,), jnp.float32)
        pltpu.sync_copy(zeros_hbm.at[pl.ds(sid * SLICE, SLICE)],
                        c_sh.at[pl.ds(sid * SLICE, SLICE)])

    plsc.subcore_barrier()

    @pl.when(cid == 0)
    def _():
        for j in range(NCHUNK):
            pltpu.sync_copy(ones_v.at[j], c_sh.at[idx_v.at[j]], add=True)

    plsc.subcore_barrier()

    @pl.when(cid == 0)
    def _():
        pltpu.sync_copy(c_sh.at[pl.ds(sid * SLICE, SLICE)],
                        out_hbm.at[0, pl.ds(sid * SLICE, SLICE)])


def _matvec_body(tab_ref, cnt_ref, w1_ref, b1_ref, w2_ref, b2_ref, o_ref,
                 acc_ref):
    i = pl.program_id(0)

    @pl.when(i == 0)
    def _():
        acc_ref[...] = jnp.zeros_like(acc_ref)

    c = cnt_ref[...]

    @pl.when(i < G - 1)
    def _():
        acc_ref[...] += lax.dot_general(c, tab_ref[...],
                                        (((1,), (1,)), ((), ())),
                                        preferred_element_type=jnp.float32)

    @pl.when(i == G - 1)
    def _():
        lane = lax.broadcasted_iota(jnp.int32, (1, B), 1) + i * B
        tb = jnp.where(lane < VOCAB, tab_ref[...], 0.0)
        acc = acc_ref[...] + lax.dot_general(c, tb, (((1,), (1,)), ((), ())),
                                             preferred_element_type=jnp.float32)
        e = acc * (1.0 / L)
        h = lax.dot_general(e, w1_ref[...], (((1,), (1,)), ((), ())),
                            preferred_element_type=jnp.float32)
        h = jnp.maximum(h + b1_ref[...], 0.0)
        o_ref[...] = jnp.sum(h * w2_ref[...], axis=1, keepdims=True) \
            + b2_ref[...]


def kernel(x, table, W1, b1, W2, b2):
    xi = x.astype(jnp.int32).reshape(NS, NCHUNK, CHUNK)
    zeros = jnp.zeros((P,), jnp.float32)
    counts = _sc_counts(xi, zeros)
    tableT = jnp.transpose(table)
    out = pl.pallas_call(
        _matvec_body,
        grid=(G,),
        in_specs=[
            pl.BlockSpec((EMBED, B), lambda i: (0, i)),
            pl.BlockSpec((1, B), lambda i: (0, i)),
            pl.BlockSpec((HIDDEN, EMBED), lambda i: (0, 0)),
            pl.BlockSpec((1, HIDDEN), lambda i: (0, 0)),
            pl.BlockSpec((1, HIDDEN), lambda i: (0, 0)),
            pl.BlockSpec((1, 1), lambda i: (0, 0)),
        ],
        out_specs=pl.BlockSpec((1, 1), lambda i: (0, 0)),
        out_shape=jax.ShapeDtypeStruct((1, 1), jnp.float32),
        scratch_shapes=[pltpu.VMEM((1, EMBED), jnp.float32)],
    )(tableT, counts, W1, b1.reshape(1, HIDDEN), W2, b2.reshape(1, 1))
    return out.reshape(1)


# speedup vs baseline: 1.1481x; 1.0052x over previous
import functools

import jax
import jax.numpy as jnp
from jax import lax
from jax.experimental import pallas as pl
from jax.experimental.pallas import tpu as pltpu
from jax.experimental.pallas import tpu_sc as plsc

VOCAB = 1000000
EMBED = 64
HIDDEN = 128
L = 16384

NC = 2
NS = 16
PER_T = L // NS
CHUNK = 128
NCHUNK = PER_T // CHUNK

B = 50176
G = 20
P = G * B
SLICE = P // NS


@functools.partial(
    pl.kernel,
    mesh=plsc.VectorSubcoreMesh(core_axis_name="c", subcore_axis_name="s"),
    out_type=jax.ShapeDtypeStruct((1, P), jnp.float32),
    scratch_types=[
        pltpu.VMEM((NCHUNK, CHUNK), jnp.int32),
        pltpu.VMEM((NCHUNK, CHUNK), jnp.float32),
        pltpu.VMEM_SHARED((P,), jnp.float32),
    ],
)
def _sc_counts(x_hbm, zeros_hbm, out_hbm, idx_v, ones_v, c_sh):
    cid = lax.axis_index("c")
    sid = lax.axis_index("s")

    @pl.when(cid == 0)
    def _():
        pltpu.sync_copy(x_hbm.at[sid], idx_v)
        for j in range(NCHUNK):
            for k in range(CHUNK // 16):
                ones_v[j, pl.ds(k * 16, 16)] = jnp.ones((16,), jnp.float32)
        pltpu.sync_copy(zeros_hbm.at[pl.ds(sid * SLICE, SLICE)],
                        c_sh.at[pl.ds(sid * SLICE, SLICE)])

    plsc.subcore_barrier()

    @pl.when(cid == 0)
    def _():
        for j in range(NCHUNK):
            pltpu.sync_copy(ones_v.at[j], c_sh.at[idx_v.at[j]], add=True)

    plsc.subcore_barrier()

    @pl.when(cid == 0)
    def _():
        pltpu.sync_copy(c_sh.at[pl.ds(sid * SLICE, SLICE)],
                        out_hbm.at[0, pl.ds(sid * SLICE, SLICE)])


def _matvec_body(tab_ref, cnt_ref, w1_ref, b1_ref, w2_ref, b2_ref, o_ref,
                 acc_ref):
    i = pl.program_id(0)

    @pl.when(i == 0)
    def _():
        acc_ref[...] = jnp.zeros_like(acc_ref)

    c = cnt_ref[...]

    @pl.when(i < G - 1)
    def _():
        acc_ref[...] += lax.dot_general(c, tab_ref[...],
                                        (((1,), (1,)), ((), ())),
                                        preferred_element_type=jnp.float32)

    @pl.when(i == G - 1)
    def _():
        lane = lax.broadcasted_iota(jnp.int32, (1, B), 1) + i * B
        tb = jnp.where(lane < VOCAB, tab_ref[...], 0.0)
        acc = acc_ref[...] + lax.dot_general(c, tb, (((1,), (1,)), ((), ())),
                                             preferred_element_type=jnp.float32)
        e = acc * (1.0 / L)
        h = lax.dot_general(e, w1_ref[...], (((1,), (1,)), ((), ())),
                            preferred_element_type=jnp.float32)
        h = jnp.maximum(h + b1_ref[...], 0.0)
        o_ref[...] = jnp.sum(h * w2_ref[...], axis=1, keepdims=True) \
            + b2_ref[...]


def kernel(x, table, W1, b1, W2, b2):
    xi = x.astype(jnp.int32).reshape(NS, NCHUNK, CHUNK)
    zeros = jnp.zeros((P,), jnp.float32)
    counts = _sc_counts(xi, zeros)
    tableT = jnp.transpose(table)
    out = pl.pallas_call(
        _matvec_body,
        grid=(G,),
        in_specs=[
            pl.BlockSpec((EMBED, B), lambda i: (0, i)),
            pl.BlockSpec((1, B), lambda i: (0, i)),
            pl.BlockSpec((HIDDEN, EMBED), lambda i: (0, 0)),
            pl.BlockSpec((1, HIDDEN), lambda i: (0, 0)),
            pl.BlockSpec((1, HIDDEN), lambda i: (0, 0)),
            pl.BlockSpec((1, 1), lambda i: (0, 0)),
        ],
        out_specs=pl.BlockSpec((1, 1), lambda i: (0, 0)),
        out_shape=jax.ShapeDtypeStruct((1, 1), jnp.float32),
        scratch_shapes=[pltpu.VMEM((1, EMBED), jnp.float32)],
    )(tableT, counts, W1, b1.reshape(1, HIDDEN), W2, b2.reshape(1, 1))
    return out.reshape(1)
